# baseline (speedup 1.0000x reference)
import functools

import jax
import jax.numpy as jnp
from jax import lax
from jax.experimental import pallas as pl
from jax.experimental.pallas import tpu as pltpu
from jax.experimental.pallas import tpu_sc as plsc

N = 10000
E = 320000
D = 128
NC = 2
NS = 16
NW = NC * NS
EPW = E // NW
C = 128
NFULL = EPW // C
TAIL = EPW - NFULL * C
ROWS_MAIN = 632
ROWS_LAST = N - 15 * ROWS_MAIN
BSEARCH_ITERS = 14


def _sc_aggregate(x, rp_pad, col, zeros_init):
    mesh = plsc.VectorSubcoreMesh(core_axis_name="c", subcore_axis_name="s")

    @functools.partial(
        pl.kernel,
        out_type=jax.ShapeDtypeStruct((NC, N, D), jnp.float32),
        mesh=mesh,
        scratch_types=[
            pltpu.VMEM((N + 8,), jnp.int32),
            pltpu.VMEM((2, C), jnp.int32),
            pltpu.VMEM((2, C), jnp.int32),
            pltpu.VMEM((2, C, D), jnp.float32),
            pltpu.VMEM((1, TAIL), jnp.int32),
            pltpu.VMEM((1, TAIL), jnp.int32),
            pltpu.VMEM((TAIL, D), jnp.float32),
            pltpu.VMEM_SHARED((N, D), jnp.float32),
            pltpu.SemaphoreType.DMA,
            pltpu.SemaphoreType.DMA,
            pltpu.SemaphoreType.DMA,
            pltpu.SemaphoreType.DMA,
            pltpu.SemaphoreType.DMA,
            pltpu.SemaphoreType.DMA,
            pltpu.SemaphoreType.DMA,
        ],
        compiler_params=pltpu.CompilerParams(needs_layout_passes=False),
    )
    def agg(x_hbm, rp_hbm, col_hbm, z_hbm, out_hbm,
            rp_v, col_v, rid_v, rows_v, col_t, rid_t, rows_t, xp_sh,
            sem, csem0, csem1, gsem0, gsem1, ssem0, ssem1):
        csem = (csem0, csem1)
        gsem = (gsem0, gsem1)
        ssem = (ssem0, ssem1)
        c = lax.axis_index("c")
        s = lax.axis_index("s")

        @pl.when(s < 15)
        def _zero_main():
            pltpu.sync_copy(z_hbm.at[pl.ds(s * ROWS_MAIN, ROWS_MAIN)],
                            xp_sh.at[pl.ds(s * ROWS_MAIN, ROWS_MAIN)])

        @pl.when(s == 15)
        def _zero_last():
            pltpu.sync_copy(z_hbm.at[pl.ds(15 * ROWS_MAIN, ROWS_LAST)],
                            xp_sh.at[pl.ds(15 * ROWS_MAIN, ROWS_LAST)])

        pltpu.sync_copy(rp_hbm, rp_v)
        plsc.subcore_barrier()

        wid = c * NS + s
        ebase = wid * EPW
        lane = lax.iota(jnp.int32, 16)

        def search16(e):
            lo = jnp.zeros((16,), jnp.int32)
            hi = jnp.full((16,), N + 1, jnp.int32)

            def step(_, lh):
                plo, phi = lh
                mid = (plo + phi) >> 1
                v = plsc.load_gather(rp_v, [mid])
                p = v <= e
                return jnp.where(p, mid, plo), jnp.where(p, phi, mid)

            lo, hi = lax.fori_loop(0, BSEARCH_ITERS, step, (lo, hi))
            return lo

        def fire_col(k, b):
            pltpu.async_copy(col_hbm.at[pl.ds(ebase + k * C, C)],
                             col_v.at[b], csem[b])

        def wait_col(b):
            pltpu.make_async_copy(col_hbm.at[pl.ds(ebase, C)],
                                  col_v.at[b], csem[b]).wait()

        def search_chunk(k, b):
            def grp(g, _):
                e = ebase + k * C + g * 16 + lane
                rid_v[b, pl.ds(g * 16, 16)] = search16(e)
                return 0

            lax.fori_loop(0, C // 16, grp, 0)

        fire_col(0, 0)
        fire_col(1, 1)

        def pair(kk, _):
            for b in range(2):
                k = kk * 2 + b

                @pl.when(kk >= 1)
                def _free_buf(b=b):
                    pltpu.make_async_copy(
                        rows_v.at[b], xp_sh.at[rid_v.at[b]], ssem[b]).wait()

                wait_col(b)
                pltpu.async_copy(x_hbm.at[col_v.at[b]], rows_v.at[b], gsem[b])
                search_chunk(k, b)
                pltpu.make_async_copy(
                    x_hbm.at[col_v.at[b]], rows_v.at[b], gsem[b]).wait()

                @pl.when(k + 2 < NFULL)
                def _prefetch_col(k=k, b=b):
                    fire_col(k + 2, b)

                pltpu.async_copy(
                    rows_v.at[b], xp_sh.at[rid_v.at[b]], ssem[b], add=True)
            return 0

        lax.fori_loop(0, NFULL // 2, pair, 0)
        for b in range(2):
            pltpu.make_async_copy(
                rows_v.at[b], xp_sh.at[rid_v.at[b]], ssem[b]).wait()

        tbase = ebase + NFULL * C
        pltpu.sync_copy(col_hbm.at[pl.ds(tbase, TAIL)], col_t.at[0])
        tail_cp = pltpu.async_copy(x_hbm.at[col_t.at[0]], rows_t, sem)
        rid_t[0, :] = search16(tbase + lane)
        tail_cp.wait()
        pltpu.sync_copy(rows_t, xp_sh.at[rid_t.at[0]], add=True)

        plsc.subcore_barrier()

        @pl.when(s < 15)
        def _write_main():
            pltpu.sync_copy(xp_sh.at[pl.ds(s * ROWS_MAIN, ROWS_MAIN)],
                            out_hbm.at[c, pl.ds(s * ROWS_MAIN, ROWS_MAIN)])

        @pl.when(s == 15)
        def _write_last():
            pltpu.sync_copy(xp_sh.at[pl.ds(15 * ROWS_MAIN, ROWS_LAST)],
                            out_hbm.at[c, pl.ds(15 * ROWS_MAIN, ROWS_LAST)])

    return agg(x, rp_pad, col, zeros_init)


def _tc_transform(partials, weights):
    blk = 1000

    def body(p_ref, w_ref, o_ref):
        acc = p_ref[0] + p_ref[1]
        o_ref[...] = jnp.dot(acc, w_ref[...],
                             preferred_element_type=jnp.float32)

    return pl.pallas_call(
        body,
        grid=(N // blk,),
        in_specs=[
            pl.BlockSpec((2, blk, D), lambda i: (0, i, 0)),
            pl.BlockSpec((D, D), lambda i: (0, 0)),
        ],
        out_specs=pl.BlockSpec((blk, D), lambda i: (i, 0)),
        out_shape=jax.ShapeDtypeStruct((N, D), jnp.float32),
    )(partials, weights)


def kernel(X, row_pointers, column_index, blockPartition, edgeToColumn,
           edgeToRow, hybrid_type, row_nzr, col_nzr, output, weights):
    rp_pad = jnp.concatenate(
        [row_pointers, jnp.full((7,), E, jnp.int32)])
    zeros_init = jnp.zeros((N, D), jnp.float32)
    partials = _sc_aggregate(X, rp_pad, column_index, zeros_init)
    return _tc_transform(partials, weights)


# speedup vs baseline: 1.0119x; 1.0119x over previous
import functools

import jax
import jax.numpy as jnp
from jax import lax
from jax.experimental import pallas as pl
from jax.experimental.pallas import tpu as pltpu
from jax.experimental.pallas import tpu_sc as plsc

N = 10000
E = 320000
D = 128
NC = 2
NS = 16
NW = NC * NS
EPW = E // NW
C = 128
NFULL = EPW // C
TAIL = EPW - NFULL * C
ROWS_MAIN = 632
ROWS_LAST = N - 15 * ROWS_MAIN
BSEARCH_ITERS = 14


def _sc_aggregate(x, rp_pad, col):
    mesh = plsc.VectorSubcoreMesh(core_axis_name="c", subcore_axis_name="s")

    @functools.partial(
        pl.kernel,
        out_type=jax.ShapeDtypeStruct((NC, N, D), jnp.float32),
        mesh=mesh,
        scratch_types=[
            pltpu.VMEM((N + 8,), jnp.int32),
            pltpu.VMEM((2, C), jnp.int32),
            pltpu.VMEM((2, C), jnp.int32),
            pltpu.VMEM((2, C, D), jnp.float32),
            pltpu.VMEM((1, TAIL), jnp.int32),
            pltpu.VMEM((1, TAIL), jnp.int32),
            pltpu.VMEM((TAIL, D), jnp.float32),
            pltpu.VMEM_SHARED((N, D), jnp.float32),
            pltpu.SemaphoreType.DMA,
            pltpu.SemaphoreType.DMA,
            pltpu.SemaphoreType.DMA,
            pltpu.SemaphoreType.DMA,
            pltpu.SemaphoreType.DMA,
            pltpu.SemaphoreType.DMA,
            pltpu.SemaphoreType.DMA,
        ],
        compiler_params=pltpu.CompilerParams(needs_layout_passes=False),
    )
    def agg(x_hbm, rp_hbm, col_hbm, out_hbm,
            rp_v, col_v, rid_v, rows_v, col_t, rid_t, rows_t, xp_sh,
            sem, csem0, csem1, gsem0, gsem1, ssem0, ssem1):
        csem = (csem0, csem1)
        gsem = (gsem0, gsem1)
        ssem = (ssem0, ssem1)
        c = lax.axis_index("c")
        s = lax.axis_index("s")

        wid = c * NS + s
        ebase = wid * EPW
        lane = lax.iota(jnp.int32, 16)
        zero16 = jnp.zeros((16,), jnp.float32)

        def zrow(i, _):
            def zcol(g, _):
                rows_v[0, i, pl.ds(g * 16, 16)] = zero16
                return 0
            lax.fori_loop(0, D // 16, zcol, 0)
            return 0

        lax.fori_loop(0, C, zrow, 0)
        rbase = s * ROWS_MAIN

        @pl.when(s < 15)
        def _zero_main():
            for off, sz in ((0, 128), (128, 128), (256, 128), (384, 128),
                            (512, 120)):
                pltpu.sync_copy(rows_v.at[0, pl.ds(0, sz)],
                                xp_sh.at[pl.ds(rbase + off, sz)])

        @pl.when(s == 15)
        def _zero_last():
            for off, sz in ((0, 128), (128, 128), (256, 128), (384, 128),
                            (512, 8)):
                pltpu.sync_copy(rows_v.at[0, pl.ds(0, sz)],
                                xp_sh.at[pl.ds(rbase + off, sz)])

        pltpu.sync_copy(rp_hbm, rp_v.at[pl.ds(0, N + 1)])
        plsc.subcore_barrier()

        def search16(e):
            lo = jnp.zeros((16,), jnp.int32)
            hi = jnp.full((16,), N + 1, jnp.int32)

            def step(_, lh):
                plo, phi = lh
                mid = (plo + phi) >> 1
                v = plsc.load_gather(rp_v, [mid])
                p = v <= e
                return jnp.where(p, mid, plo), jnp.where(p, phi, mid)

            lo, hi = lax.fori_loop(0, BSEARCH_ITERS, step, (lo, hi))
            return lo

        def fire_col(k, b):
            pltpu.async_copy(col_hbm.at[pl.ds(ebase + k * C, C)],
                             col_v.at[b], csem[b])

        def wait_col(b):
            pltpu.make_async_copy(col_hbm.at[pl.ds(ebase, C)],
                                  col_v.at[b], csem[b]).wait()

        def search_chunk(k, b):
            def grp(g, _):
                e = ebase + k * C + g * 16 + lane
                rid_v[b, pl.ds(g * 16, 16)] = search16(e)
                return 0

            lax.fori_loop(0, C // 16, grp, 0)

        fire_col(0, 0)
        fire_col(1, 1)
        tbase = ebase + NFULL * C
        pltpu.async_copy(col_hbm.at[pl.ds(tbase, TAIL)], col_t.at[0], sem)

        def pair(kk, _):
            for b in range(2):
                k = kk * 2 + b

                @pl.when(kk >= 1)
                def _free_buf(b=b):
                    pltpu.make_async_copy(
                        rows_v.at[b], xp_sh.at[rid_v.at[b]], ssem[b]).wait()

                wait_col(b)
                pltpu.async_copy(x_hbm.at[col_v.at[b]], rows_v.at[b], gsem[b])
                search_chunk(k, b)
                pltpu.make_async_copy(
                    x_hbm.at[col_v.at[b]], rows_v.at[b], gsem[b]).wait()

                @pl.when(k + 2 < NFULL)
                def _prefetch_col(k=k, b=b):
                    fire_col(k + 2, b)

                pltpu.async_copy(
                    rows_v.at[b], xp_sh.at[rid_v.at[b]], ssem[b], add=True)
            return 0

        lax.fori_loop(0, NFULL // 2, pair, 0)
        for b in range(2):
            pltpu.make_async_copy(
                rows_v.at[b], xp_sh.at[rid_v.at[b]], ssem[b]).wait()

        pltpu.make_async_copy(col_hbm.at[pl.ds(tbase, TAIL)], col_t.at[0],
                              sem).wait()
        tail_cp = pltpu.async_copy(x_hbm.at[col_t.at[0]], rows_t, sem)
        rid_t[0, :] = search16(tbase + lane)
        tail_cp.wait()
        pltpu.sync_copy(rows_t, xp_sh.at[rid_t.at[0]], add=True)

        plsc.subcore_barrier()

        @pl.when(s < 15)
        def _write_main():
            pltpu.sync_copy(xp_sh.at[pl.ds(s * ROWS_MAIN, ROWS_MAIN)],
                            out_hbm.at[c, pl.ds(s * ROWS_MAIN, ROWS_MAIN)])

        @pl.when(s == 15)
        def _write_last():
            pltpu.sync_copy(xp_sh.at[pl.ds(15 * ROWS_MAIN, ROWS_LAST)],
                            out_hbm.at[c, pl.ds(15 * ROWS_MAIN, ROWS_LAST)])

    return agg(x, rp_pad, col)


def _tc_transform(partials, weights):
    blk = 1000

    def body(p_ref, w_ref, o_ref):
        acc = p_ref[0] + p_ref[1]
        o_ref[...] = jnp.dot(acc, w_ref[...],
                             preferred_element_type=jnp.float32)

    return pl.pallas_call(
        body,
        grid=(N // blk,),
        in_specs=[
            pl.BlockSpec((2, blk, D), lambda i: (0, i, 0)),
            pl.BlockSpec((D, D), lambda i: (0, 0)),
        ],
        out_specs=pl.BlockSpec((blk, D), lambda i: (i, 0)),
        out_shape=jax.ShapeDtypeStruct((N, D), jnp.float32),
    )(partials, weights)


def kernel(X, row_pointers, column_index, blockPartition, edgeToColumn,
           edgeToRow, hybrid_type, row_nzr, col_nzr, output, weights):
    partials = _sc_aggregate(X, row_pointers, column_index)
    return _tc_transform(partials, weights)


# speedup vs baseline: 1.0332x; 1.0210x over previous
import functools

import jax
import jax.numpy as jnp
from jax import lax
from jax.experimental import pallas as pl
from jax.experimental.pallas import tpu as pltpu
from jax.experimental.pallas import tpu_sc as plsc

N = 10000
E = 320000
D = 128
NC = 2
NS = 16
NW = NC * NS
EPW = E // NW
C = 128
NFULL = EPW // C
TAIL = EPW - NFULL * C
ROWS_MAIN = 632
ROWS_LAST = N - 15 * ROWS_MAIN
BSEARCH_ITERS = 14


def _sc_aggregate(x, rp_pad, col):
    mesh = plsc.VectorSubcoreMesh(core_axis_name="c", subcore_axis_name="s")

    @functools.partial(
        pl.kernel,
        out_type=jax.ShapeDtypeStruct((NC, N, D), jnp.float32),
        mesh=mesh,
        scratch_types=[
            pltpu.VMEM((N + 8,), jnp.int32),
            pltpu.VMEM((2, C), jnp.int32),
            pltpu.VMEM((2, C), jnp.int32),
            pltpu.VMEM((2, C, D), jnp.float32),
            pltpu.VMEM((1, TAIL), jnp.int32),
            pltpu.VMEM((1, TAIL), jnp.int32),
            pltpu.VMEM((TAIL, D), jnp.float32),
            pltpu.VMEM_SHARED((N, D), jnp.float32),
            pltpu.SemaphoreType.DMA,
            pltpu.SemaphoreType.DMA,
            pltpu.SemaphoreType.DMA,
            pltpu.SemaphoreType.DMA,
            pltpu.SemaphoreType.DMA,
            pltpu.SemaphoreType.DMA,
            pltpu.SemaphoreType.DMA,
        ],
        compiler_params=pltpu.CompilerParams(needs_layout_passes=False),
    )
    def agg(x_hbm, rp_hbm, col_hbm, out_hbm,
            rp_v, col_v, rid_v, rows_v, col_t, rid_t, rows_t, xp_sh,
            sem, csem0, csem1, gsem0, gsem1, ssem0, ssem1):
        csem = (csem0, csem1)
        gsem = (gsem0, gsem1)
        ssem = (ssem0, ssem1)
        c = lax.axis_index("c")
        s = lax.axis_index("s")

        wid = c * NS + s
        ebase = wid * EPW
        lane = lax.iota(jnp.int32, 16)
        zero16 = jnp.zeros((16,), jnp.float32)

        def zrow(i, _):
            def zcol(g, _):
                rows_v[0, i, pl.ds(g * 16, 16)] = zero16
                return 0
            lax.fori_loop(0, D // 16, zcol, 0)
            return 0

        lax.fori_loop(0, C, zrow, 0)
        rbase = s * ROWS_MAIN

        @pl.when(s < 15)
        def _zero_main():
            for off, sz in ((0, 128), (128, 128), (256, 128), (384, 128),
                            (512, 120)):
                pltpu.sync_copy(rows_v.at[0, pl.ds(0, sz)],
                                xp_sh.at[pl.ds(rbase + off, sz)])

        @pl.when(s == 15)
        def _zero_last():
            for off, sz in ((0, 128), (128, 128), (256, 128), (384, 128),
                            (512, 8)):
                pltpu.sync_copy(rows_v.at[0, pl.ds(0, sz)],
                                xp_sh.at[pl.ds(rbase + off, sz)])

        pltpu.sync_copy(rp_hbm, rp_v.at[pl.ds(0, N + 1)])
        plsc.subcore_barrier()

        def search16(e):
            lo = jnp.zeros((16,), jnp.int32)
            hi = jnp.full((16,), N + 1, jnp.int32)

            def step(_, lh):
                plo, phi = lh
                mid = (plo + phi) >> 1
                v = plsc.load_gather(rp_v, [mid])
                p = v <= e
                return jnp.where(p, mid, plo), jnp.where(p, phi, mid)

            lo, hi = lax.fori_loop(0, BSEARCH_ITERS, step, (lo, hi))
            return lo

        def fire_col(k, b):
            pltpu.async_copy(col_hbm.at[pl.ds(ebase + k * C, C)],
                             col_v.at[b], csem[b])

        def wait_col(b):
            pltpu.make_async_copy(col_hbm.at[pl.ds(ebase, C)],
                                  col_v.at[b], csem[b]).wait()

        def search_chunk(k, b):
            def grp(g, _):
                e = ebase + k * C + g * 16 + lane
                rid_v[b, pl.ds(g * 16, 16)] = search16(e)
                return 0

            lax.fori_loop(0, C // 16, grp, 0)

        fire_col(0, 0)
        fire_col(1, 1)
        tbase = ebase + NFULL * C
        pltpu.async_copy(col_hbm.at[pl.ds(tbase, TAIL)], col_t.at[0], sem)

        def pair(kk, _):
            for b in range(2):
                k = kk * 2 + b

                @pl.when(kk >= 1)
                def _free_buf(b=b):
                    pltpu.make_async_copy(
                        rows_v.at[b], xp_sh.at[rid_v.at[b]], ssem[b]).wait()

                wait_col(b)
                pltpu.async_copy(x_hbm.at[col_v.at[b]], rows_v.at[b], gsem[b])
                search_chunk(k, b)
                pltpu.make_async_copy(
                    x_hbm.at[col_v.at[b]], rows_v.at[b], gsem[b]).wait()

                @pl.when(k + 2 < NFULL)
                def _prefetch_col(k=k, b=b):
                    fire_col(k + 2, b)

                pltpu.async_copy(
                    rows_v.at[b], xp_sh.at[rid_v.at[b]], ssem[b], add=True)
            return 0

        lax.fori_loop(0, NFULL // 2, pair, 0)
        for b in range(2):
            pltpu.make_async_copy(
                rows_v.at[b], xp_sh.at[rid_v.at[b]], ssem[b]).wait()

        pltpu.make_async_copy(col_hbm.at[pl.ds(tbase, TAIL)], col_t.at[0],
                              sem).wait()
        tail_cp = pltpu.async_copy(x_hbm.at[col_t.at[0]], rows_t, sem)
        rid_t[0, :] = search16(tbase + lane)
        tail_cp.wait()
        pltpu.sync_copy(rows_t, xp_sh.at[rid_t.at[0]], add=True)

        plsc.subcore_barrier()

        @pl.when(s < 15)
        def _write_main():
            pltpu.sync_copy(xp_sh.at[pl.ds(s * ROWS_MAIN, ROWS_MAIN)],
                            out_hbm.at[c, pl.ds(s * ROWS_MAIN, ROWS_MAIN)])

        @pl.when(s == 15)
        def _write_last():
            pltpu.sync_copy(xp_sh.at[pl.ds(15 * ROWS_MAIN, ROWS_LAST)],
                            out_hbm.at[c, pl.ds(15 * ROWS_MAIN, ROWS_LAST)])

    return agg(x, rp_pad, col)


def _tc_transform(partials, weights):
    blk = 2000

    def body(p_ref, w_ref, o_ref):
        acc = p_ref[0] + p_ref[1]
        o_ref[...] = jnp.dot(acc, w_ref[...],
                             preferred_element_type=jnp.float32)

    return pl.pallas_call(
        body,
        grid=(N // blk,),
        in_specs=[
            pl.BlockSpec((2, blk, D), lambda i: (0, i, 0)),
            pl.BlockSpec((D, D), lambda i: (0, 0)),
        ],
        out_specs=pl.BlockSpec((blk, D), lambda i: (i, 0)),
        out_shape=jax.ShapeDtypeStruct((N, D), jnp.float32),
    )(partials, weights)


def kernel(X, row_pointers, column_index, blockPartition, edgeToColumn,
           edgeToRow, hybrid_type, row_nzr, col_nzr, output, weights):
    partials = _sc_aggregate(X, row_pointers, column_index)
    return _tc_transform(partials, weights)
